# Initial kernel scaffold; baseline (speedup 1.0000x reference)
import jax
import jax.numpy as jnp
from jax.experimental import pallas as pl


def kernel(x, masks, W1, b1, W2, b2):
    raise NotImplementedError("write your pallas kernel here")


# speedup vs baseline: 2.3197x; 2.3197x over previous
import functools

import jax
import jax.numpy as jnp
import numpy as np
from jax.experimental import pallas as pl

_TM = 256
_SQRT_HALF = np.float32(0.7071067811865476)


def _mlp_body(x_ref, w1_ref, b1_ref, w2_ref, b2_ref, out_ref):
    h = jnp.dot(x_ref[...], w1_ref[...], preferred_element_type=jnp.float32)
    h = h + b1_ref[...]
    g = h * (0.5 * (1.0 + jax.lax.erf(h * _SQRT_HALF)))
    acc = jnp.dot(g.astype(jnp.bfloat16), w2_ref[...],
                  preferred_element_type=jnp.float32)
    out_ref[...] = acc + b2_ref[...]


@functools.partial(jax.jit, static_argnums=())
def kernel(x, masks, W1, b1, W2, b2):
    B, S, D_in = x.shape
    D_out = W1.shape[1]
    M = B * S
    xm = x.reshape(M, D_in).astype(jnp.bfloat16)
    w1 = W1.astype(jnp.bfloat16)
    w2 = W2.astype(jnp.bfloat16)
    b1r = b1.reshape(1, D_out)
    b2r = b2.reshape(1, D_out)

    num_m = M // _TM
    out = pl.pallas_call(
        _mlp_body,
        grid=(num_m,),
        in_specs=[
            pl.BlockSpec((_TM, D_in), lambda m: (m, 0)),
            pl.BlockSpec((D_in, D_out), lambda m: (0, 0)),
            pl.BlockSpec((1, D_out), lambda m: (0, 0)),
            pl.BlockSpec((D_out, D_out), lambda m: (0, 0)),
            pl.BlockSpec((1, D_out), lambda m: (0, 0)),
        ],
        out_specs=pl.BlockSpec((_TM, D_out), lambda m: (m, 0)),
        out_shape=jax.ShapeDtypeStruct((M, D_out), jnp.float32),
    )(xm, w1, b1r, w2, b2r)
    return (out.reshape(B, S, D_out), masks)
